# baseline (speedup 1.0000x reference)
import jax
import jax.numpy as jnp
from jax import lax
from jax.experimental import pallas as pl
from jax.experimental.pallas import tpu as pltpu
from jax.experimental.pallas import tpu_sc as plsc

_LANES = 16


def _sc_gather_body(table_hbm, idx_hbm, out_hbm, table_v, idx_v, out_v):
    head = lax.axis_index("s")
    half = lax.axis_index("c")
    row = table_v.shape[0]
    chunk = idx_v.shape[0]
    p_pad = 2 * chunk
    pltpu.sync_copy(idx_hbm.at[pl.ds(half * chunk, chunk)], idx_v)
    pltpu.sync_copy(table_hbm.at[pl.ds(head * row, row)], table_v)

    def body(i, carry):
        sl = pl.ds(i * _LANES, _LANES)
        out_v[sl] = plsc.load_gather(table_v, [idx_v[sl]])
        return carry

    lax.fori_loop(0, chunk // _LANES, body, 0)
    pltpu.sync_copy(out_v, out_hbm.at[pl.ds(head * p_pad + half * chunk, chunk)])


def _sc_gather(table_flat, idx_flat_pad, num_heads, row, p_pad):
    chunk = p_pad // 2
    mesh = plsc.VectorSubcoreMesh(core_axis_name="c", subcore_axis_name="s")
    return pl.kernel(
        _sc_gather_body,
        out_type=jax.ShapeDtypeStruct((num_heads * p_pad,), jnp.float32),
        mesh=mesh,
        compiler_params=pltpu.CompilerParams(
            needs_layout_passes=False, use_tc_tiling_on_sc=True
        ),
        scratch_types=[
            pltpu.VMEM((row,), jnp.float32),
            pltpu.VMEM((chunk,), jnp.int32),
            pltpu.VMEM((chunk,), jnp.float32),
        ],
    )(table_flat, idx_flat_pad)


def _add_body(x_ref, pos_ref, o_ref):
    o_ref[...] = x_ref[...] + pos_ref[...]


def kernel(inputs, relative_position_bias_table, relative_position_index):
    b, h, s_q, s_k = inputs.shape
    p = s_q * s_k
    p_pad = ((p + 255) // 256) * 256

    idx = relative_position_index[:s_q, :s_k].astype(jnp.int32).reshape(-1)
    idx_pad = jnp.pad(idx, (0, p_pad - p))
    nrd = relative_position_bias_table.shape[1]
    row = ((nrd + 127) // 128) * 128
    table_flat = jnp.pad(
        relative_position_bias_table, ((0, 0), (0, row - nrd))
    ).reshape(-1)

    pos_flat = _sc_gather(table_flat, idx_pad, h, row, p_pad)
    pos = pos_flat.reshape(h, p_pad)[:, :p]

    x = inputs.reshape(b, h, p)
    out = pl.pallas_call(
        _add_body,
        out_shape=jax.ShapeDtypeStruct((b, h, p), jnp.float32),
        grid=(b,),
        in_specs=[
            pl.BlockSpec((1, h, p), lambda i: (i, 0, 0)),
            pl.BlockSpec((h, p), lambda i: (0, 0)),
        ],
        out_specs=pl.BlockSpec((1, h, p), lambda i: (i, 0, 0)),
    )(x, pos)
    return out.reshape(b, h, s_q, s_k)


# speedup vs baseline: 3.7511x; 3.7511x over previous
import jax
import jax.numpy as jnp
from jax import lax
from jax.experimental import pallas as pl
from jax.experimental.pallas import tpu as pltpu
from jax.experimental.pallas import tpu_sc as plsc

_LANES = 16
_WORKERS = 32


def _make_sc_body(num_heads, s_k, tasks_per_worker, k_starts):
    h_shift = num_heads.bit_length() - 1
    h_mask = num_heads - 1

    def body(table_hbm, idx_hbm, out_hbm, table_v, idx_v, out_v):
        wid = lax.axis_index("s") * 2 + lax.axis_index("c")
        pltpu.sync_copy(table_hbm, table_v)
        pltpu.sync_copy(idx_hbm, idx_v)

        t0 = wid * tasks_per_worker

        def task(tl, carry):
            t = t0 + tl
            q = lax.shift_right_logical(t, h_shift)
            h = lax.bitwise_and(t, h_mask)
            idx_base = q * s_k
            tab_base = h * (table_v.shape[0] // num_heads)
            for k0 in k_starts:
                iv = idx_v[pl.ds(idx_base + k0, _LANES)] + tab_base
                out_v[tl, pl.ds(k0, _LANES)] = plsc.load_gather(table_v, [iv])
            return carry

        lax.fori_loop(0, tasks_per_worker, task, 0)
        pltpu.sync_copy(
            out_v, out_hbm.at[pl.ds(t0, tasks_per_worker), :]
        )

    return body


def _sc_gather(table_flat, idx_flat, num_heads, s_k, tasks_per_worker, k_starts):
    n_rows = _WORKERS * tasks_per_worker
    mesh = plsc.VectorSubcoreMesh(core_axis_name="c", subcore_axis_name="s")
    return pl.kernel(
        _make_sc_body(num_heads, s_k, tasks_per_worker, k_starts),
        out_type=jax.ShapeDtypeStruct((n_rows, s_k), jnp.float32),
        mesh=mesh,
        compiler_params=pltpu.CompilerParams(
            needs_layout_passes=False, use_tc_tiling_on_sc=True
        ),
        scratch_types=[
            pltpu.VMEM((table_flat.shape[0],), jnp.float32),
            pltpu.VMEM((idx_flat.shape[0],), jnp.int32),
            pltpu.VMEM((tasks_per_worker, s_k), jnp.float32),
        ],
    )(table_flat, idx_flat)


def _add_body(x_ref, pos_ref, o_ref):
    o_ref[...] = x_ref[...] + pos_ref[...]


def kernel(inputs, relative_position_bias_table, relative_position_index):
    b, h, s_q, s_k = inputs.shape

    n_tasks = s_q * h
    tasks_per_worker = -(-n_tasks // _WORKERS)
    tasks_per_worker = ((tasks_per_worker + 7) // 8) * 8
    n_rows = _WORKERS * tasks_per_worker
    q_max = (n_rows - 1) // h

    k_starts = list(range(0, s_k - _LANES + 1, _LANES))
    if k_starts[-1] + _LANES < s_k:
        k_starts.append(s_k - _LANES)

    idx = relative_position_index[:s_q, :s_k].astype(jnp.int32).reshape(-1)
    idx_len = ((q_max * s_k + k_starts[-1] + _LANES + 127) // 128) * 128
    idx_pad = jnp.pad(idx, (0, idx_len - idx.size))
    nrd = relative_position_bias_table.shape[1]
    row = ((nrd + 127) // 128) * 128
    table_flat = jnp.pad(
        relative_position_bias_table, ((0, 0), (0, row - nrd))
    ).reshape(-1)

    pos2d = _sc_gather(table_flat, idx_pad, h, s_k, tasks_per_worker, k_starts)
    pos = pos2d[:n_tasks].reshape(s_q, h, s_k)

    xt = jnp.transpose(inputs, (0, 2, 1, 3))
    out = pl.pallas_call(
        _add_body,
        out_shape=jax.ShapeDtypeStruct((b, s_q, h, s_k), jnp.float32),
        grid=(b,),
        in_specs=[
            pl.BlockSpec((1, s_q, h, s_k), lambda i: (i, 0, 0, 0)),
            pl.BlockSpec((s_q, h, s_k), lambda i: (0, 0, 0)),
        ],
        out_specs=pl.BlockSpec((1, s_q, h, s_k), lambda i: (i, 0, 0, 0)),
    )(xt, pos)
    return jnp.transpose(out, (0, 2, 1, 3))


# speedup vs baseline: 3.8381x; 1.0232x over previous
import jax
import jax.numpy as jnp
from jax import lax
from jax.experimental import pallas as pl
from jax.experimental.pallas import tpu as pltpu
from jax.experimental.pallas import tpu_sc as plsc

_LANES = 16
_WORKERS = 32


def _make_sc_body(num_heads, s_k, tasks_per_worker, k_starts):
    h_shift = num_heads.bit_length() - 1
    h_mask = num_heads - 1

    def body(table_hbm, idx_hbm, out_hbm, table_v, idx_v, out_v):
        wid = lax.axis_index("s") * 2 + lax.axis_index("c")
        pltpu.sync_copy(table_hbm, table_v)
        pltpu.sync_copy(idx_hbm, idx_v)

        t0 = wid * tasks_per_worker

        def task(tl, carry):
            t = t0 + tl
            q = lax.shift_right_logical(t, h_shift)
            h = lax.bitwise_and(t, h_mask)
            idx_base = q * s_k
            tab_base = h * (table_v.shape[0] // num_heads)
            for k0 in k_starts:
                iv = idx_v[pl.ds(idx_base + k0, _LANES)] + tab_base
                out_v[tl, pl.ds(k0, _LANES)] = plsc.load_gather(table_v, [iv])
            return carry

        lax.fori_loop(0, tasks_per_worker, task, 0)
        pltpu.sync_copy(
            out_v, out_hbm.at[pl.ds(t0, tasks_per_worker), :]
        )

    return body


def _sc_gather(table_flat, idx_flat, num_heads, s_k, tasks_per_worker, k_starts):
    n_rows = _WORKERS * tasks_per_worker
    mesh = plsc.VectorSubcoreMesh(core_axis_name="c", subcore_axis_name="s")
    return pl.kernel(
        _make_sc_body(num_heads, s_k, tasks_per_worker, k_starts),
        out_type=jax.ShapeDtypeStruct((n_rows, s_k), jnp.float32),
        mesh=mesh,
        compiler_params=pltpu.CompilerParams(
            needs_layout_passes=False, use_tc_tiling_on_sc=True
        ),
        scratch_types=[
            pltpu.VMEM((table_flat.shape[0],), jnp.float32),
            pltpu.VMEM((idx_flat.shape[0],), jnp.int32),
            pltpu.VMEM((tasks_per_worker, s_k), jnp.float32),
        ],
    )(table_flat, idx_flat)


def _add_body(x_ref, pos_ref, o_ref):
    o_ref[...] = x_ref[...] + pos_ref[...]


def kernel(inputs, relative_position_bias_table, relative_position_index):
    b, h, s_q, s_k = inputs.shape

    n_tasks = s_q * h
    tasks_per_worker = -(-n_tasks // _WORKERS)
    tasks_per_worker = ((tasks_per_worker + 7) // 8) * 8
    n_rows = _WORKERS * tasks_per_worker
    q_max = (n_rows - 1) // h

    k_starts = list(range(0, s_k - _LANES + 1, _LANES))
    if k_starts[-1] + _LANES < s_k:
        k_starts.append(s_k - _LANES)

    idx = relative_position_index[:s_q, :s_k].astype(jnp.int32).reshape(-1)
    idx_len = ((q_max * s_k + k_starts[-1] + _LANES + 127) // 128) * 128
    idx_pad = jnp.pad(idx, (0, idx_len - idx.size))
    nrd = relative_position_bias_table.shape[1]
    row = ((nrd + 127) // 128) * 128
    table_flat = jnp.pad(
        relative_position_bias_table, ((0, 0), (0, row - nrd))
    ).reshape(-1)

    pos2d = _sc_gather(table_flat, idx_pad, h, s_k, tasks_per_worker, k_starts)
    pos = pos2d[:n_tasks].reshape(s_q, h, s_k)

    xt = jnp.transpose(inputs, (0, 2, 1, 3))
    out = pl.pallas_call(
        _add_body,
        out_shape=jax.ShapeDtypeStruct((b, s_q, h, s_k), jnp.float32),
        grid=(b // 2,),
        in_specs=[
            pl.BlockSpec((2, s_q, h, s_k), lambda i: (i, 0, 0, 0)),
            pl.BlockSpec((s_q, h, s_k), lambda i: (0, 0, 0)),
        ],
        out_specs=pl.BlockSpec((2, s_q, h, s_k), lambda i: (i, 0, 0, 0)),
    )(xt, pos)
    return jnp.transpose(out, (0, 2, 1, 3))


# speedup vs baseline: 3.8710x; 1.0086x over previous
import jax
import jax.numpy as jnp
from jax import lax
from jax.experimental import pallas as pl
from jax.experimental.pallas import tpu as pltpu
from jax.experimental.pallas import tpu_sc as plsc

_LANES = 16
_WORKERS = 32


def _make_sc_body(num_heads, s_k, tasks_per_worker, k_starts):
    h_shift = num_heads.bit_length() - 1
    h_mask = num_heads - 1

    def body(table_hbm, idx_hbm, out_hbm, table_v, idx_v, out_v):
        wid = lax.axis_index("s") * 2 + lax.axis_index("c")
        pltpu.sync_copy(table_hbm, table_v)
        pltpu.sync_copy(idx_hbm, idx_v)

        t0 = wid * tasks_per_worker

        def task(tl, carry):
            t = t0 + tl
            q = lax.shift_right_logical(t, h_shift)
            h = lax.bitwise_and(t, h_mask)
            idx_base = q * s_k
            tab_base = h * (table_v.shape[0] // num_heads)
            for k0 in k_starts:
                iv = idx_v[pl.ds(idx_base + k0, _LANES)] + tab_base
                out_v[tl, pl.ds(k0, _LANES)] = plsc.load_gather(table_v, [iv])
            return carry

        lax.fori_loop(0, tasks_per_worker, task, 0)
        pltpu.sync_copy(
            out_v, out_hbm.at[pl.ds(t0, tasks_per_worker), :]
        )

    return body


def _sc_gather(table_flat, idx_flat, num_heads, s_k, tasks_per_worker, k_starts):
    n_rows = _WORKERS * tasks_per_worker
    mesh = plsc.VectorSubcoreMesh(core_axis_name="c", subcore_axis_name="s")
    return pl.kernel(
        _make_sc_body(num_heads, s_k, tasks_per_worker, k_starts),
        out_type=jax.ShapeDtypeStruct((n_rows, s_k), jnp.float32),
        mesh=mesh,
        compiler_params=pltpu.CompilerParams(
            needs_layout_passes=False, use_tc_tiling_on_sc=True
        ),
        scratch_types=[
            pltpu.VMEM((table_flat.shape[0],), jnp.float32),
            pltpu.VMEM((idx_flat.shape[0],), jnp.int32),
            pltpu.VMEM((tasks_per_worker, s_k), jnp.float32),
        ],
    )(table_flat, idx_flat)


def _add_body(x_ref, pos_ref, o_ref):
    o_ref[...] = x_ref[...] + pos_ref[...]


def kernel(inputs, relative_position_bias_table, relative_position_index):
    b, h, s_q, s_k = inputs.shape

    n_tasks = s_q * h
    tasks_per_worker = -(-n_tasks // _WORKERS)
    tasks_per_worker = ((tasks_per_worker + 7) // 8) * 8
    n_rows = _WORKERS * tasks_per_worker
    q_max = (n_rows - 1) // h

    k_starts = list(range(0, s_k - _LANES + 1, _LANES))
    if k_starts[-1] + _LANES < s_k:
        k_starts.append(s_k - _LANES)

    idx = relative_position_index[:s_q, :s_k].astype(jnp.int32).reshape(-1)
    idx_len = ((q_max * s_k + k_starts[-1] + _LANES + 127) // 128) * 128
    idx_pad = jnp.pad(idx, (0, idx_len - idx.size))
    nrd = relative_position_bias_table.shape[1]
    row = ((nrd + 127) // 128) * 128
    table_flat = jnp.pad(
        relative_position_bias_table, ((0, 0), (0, row - nrd))
    ).reshape(-1)

    pos2d = _sc_gather(table_flat, idx_pad, h, s_k, tasks_per_worker, k_starts)
    pos = pos2d[:n_tasks].reshape(s_q, h, s_k)

    xt = jnp.transpose(inputs, (0, 2, 1, 3))
    out = pl.pallas_call(
        _add_body,
        out_shape=jax.ShapeDtypeStruct((b, s_q, h, s_k), jnp.float32),
        grid=(b // 4,),
        in_specs=[
            pl.BlockSpec((4, s_q, h, s_k), lambda i: (i, 0, 0, 0)),
            pl.BlockSpec((s_q, h, s_k), lambda i: (0, 0, 0)),
        ],
        out_specs=pl.BlockSpec((4, s_q, h, s_k), lambda i: (i, 0, 0, 0)),
    )(xt, pos)
    return jnp.transpose(out, (0, 2, 1, 3))


# speedup vs baseline: 4.2840x; 1.1067x over previous
import jax
import jax.numpy as jnp
from jax import lax
from jax.experimental import pallas as pl
from jax.experimental.pallas import tpu as pltpu
from jax.experimental.pallas import tpu_sc as plsc

_LANES = 16
_WORKERS = 32


def _make_sc_body(num_heads, s_k, q_per_worker, k_starts, win):
    def body(table_hbm, idx_hbm, out_hbm, table_v, idx_v, out_v):
        wid = lax.axis_index("s") * 2 + lax.axis_index("c")
        row = table_v.shape[0] // num_heads
        q0s = wid * (q_per_worker * s_k)
        a0 = pl.multiple_of(
            lax.shift_left(lax.shift_right_logical(q0s, 7), 7), 128
        )
        shift = q0s - a0
        pltpu.sync_copy(table_hbm, table_v)
        pltpu.sync_copy(idx_hbm.at[pl.ds(a0, win)], idx_v)

        def q_loop(qi, carry):
            idx_base = shift + qi * s_k
            for k0 in k_starts:
                iv = idx_v[pl.ds(idx_base + k0, _LANES)]
                for hh in range(num_heads):
                    out_v[qi * num_heads + hh, pl.ds(k0, _LANES)] = (
                        plsc.load_gather(table_v, [iv + hh * row])
                    )
            return carry

        lax.fori_loop(0, q_per_worker, q_loop, 0)
        t0 = wid * (q_per_worker * num_heads)
        pltpu.sync_copy(
            out_v, out_hbm.at[pl.ds(t0, q_per_worker * num_heads), :]
        )

    return body


def _sc_gather(table_flat, idx_flat, num_heads, s_k, q_per_worker, k_starts, win):
    n_rows = _WORKERS * q_per_worker * num_heads
    mesh = plsc.VectorSubcoreMesh(core_axis_name="c", subcore_axis_name="s")
    return pl.kernel(
        _make_sc_body(num_heads, s_k, q_per_worker, k_starts, win),
        out_type=jax.ShapeDtypeStruct((n_rows, s_k), jnp.float32),
        mesh=mesh,
        compiler_params=pltpu.CompilerParams(
            needs_layout_passes=False, use_tc_tiling_on_sc=True
        ),
        scratch_types=[
            pltpu.VMEM((table_flat.shape[0],), jnp.float32),
            pltpu.VMEM((win,), jnp.int32),
            pltpu.VMEM((q_per_worker * num_heads, s_k), jnp.float32),
        ],
    )(table_flat, idx_flat)


def _add_body(x_ref, pos_ref, o_ref):
    o_ref[...] = x_ref[...] + pos_ref[...]


def kernel(inputs, relative_position_bias_table, relative_position_index):
    b, h, s_q, s_k = inputs.shape

    n_tasks = s_q * h
    q_per_worker = -(-s_q // _WORKERS)

    k_starts = list(range(0, s_k - _LANES + 1, _LANES))
    if k_starts[-1] + _LANES < s_k:
        k_starts.append(s_k - _LANES)

    span = 128 + (q_per_worker - 1) * s_k + k_starts[-1] + _LANES
    win = ((span + 127) // 128) * 128
    a0_max = ((_WORKERS - 1) * q_per_worker * s_k) >> 7 << 7
    idx_len = a0_max + win

    idx = relative_position_index[:s_q, :s_k].astype(jnp.int32).reshape(-1)
    idx_pad = jnp.pad(idx, (0, idx_len - idx.size))
    nrd = relative_position_bias_table.shape[1]
    row = ((nrd + 127) // 128) * 128
    table_flat = jnp.pad(
        relative_position_bias_table, ((0, 0), (0, row - nrd))
    ).reshape(-1)

    pos2d = _sc_gather(
        table_flat, idx_pad, h, s_k, q_per_worker, k_starts, win
    )
    pos = pos2d[:n_tasks].reshape(s_q, h, s_k)

    xt = jnp.transpose(inputs, (0, 2, 1, 3))
    out = pl.pallas_call(
        _add_body,
        out_shape=jax.ShapeDtypeStruct((b, s_q, h, s_k), jnp.float32),
        grid=(b // 4,),
        in_specs=[
            pl.BlockSpec((4, s_q, h, s_k), lambda i: (i, 0, 0, 0)),
            pl.BlockSpec((s_q, h, s_k), lambda i: (0, 0, 0)),
        ],
        out_specs=pl.BlockSpec((4, s_q, h, s_k), lambda i: (i, 0, 0, 0)),
    )(xt, pos)
    return jnp.transpose(out, (0, 2, 1, 3))
